# baseline (speedup 1.0000x reference)
import functools

import jax
import jax.numpy as jnp
from jax import lax
from jax.experimental import pallas as pl
from jax.experimental.pallas import tpu as pltpu, tpu_sc as plsc

N = 10000
E = 320000
D = 128
DH = D // 2
NPAD = 10240
NC, NS = 2, 16
KR = 400
NR = 5
KSB = NR * KR
NSB = E // NS // KSB
EROWS = E // KR
RPT = NPAD // NS
WB = 320

_mesh = plsc.VectorSubcoreMesh(core_axis_name="c", subcore_axis_name="s",
                               num_cores=NC, num_subcores=NS)


def _splat(v16, j):
    return lax.gather(
        v16, jnp.full((16, 1), j, jnp.int32),
        dimension_numbers=lax.GatherDimensionNumbers(
            offset_dims=(), collapsed_slice_dims=(0,), start_index_map=(0,)),
        slice_sizes=(1,),
        mode=lax.GatherScatterMode.PROMISE_IN_BOUNDS)


@functools.partial(
    pl.kernel,
    out_type=jax.ShapeDtypeStruct((NC, NPAD, DH), jnp.float32),
    mesh=_mesh,
    scratch_types=dict(
        sb2=[dict(sidx=pltpu.VMEM((NR, KR), jnp.int32),
                  didx=pltpu.VMEM((NR, KR), jnp.int32),
                  va=pltpu.VMEM((NR, KR), jnp.float32),
                  vb=pltpu.VMEM((NR, KR), jnp.float32),
                  vd=pltpu.VMEM((NR, KR), jnp.float32),
                  al=pltpu.VMEM((NR, KR), jnp.float32),
                  si=pltpu.SemaphoreType.DMA,
                  sa=pltpu.SemaphoreType.DMA,
                  sb=pltpu.SemaphoreType.DMA,
                  sd=pltpu.SemaphoreType.DMA,
                  sp=pltpu.SemaphoreType.DMA) for _ in range(2)],
        rows=[pltpu.VMEM((KR, DH), jnp.float32) for _ in range(2)],
        semr=[pltpu.SemaphoreType.DMA for _ in range(2)],
        ss=[pltpu.SemaphoreType.DMA for _ in range(2)],
        zb1=pltpu.VMEM((RPT,), jnp.float32),
        den_sh=pltpu.VMEM_SHARED((NPAD,), jnp.float32),
        acc_sh=pltpu.VMEM_SHARED((NPAD, DH), jnp.float32),
    ),
    compiler_params=pltpu.CompilerParams(use_tc_tiling_on_sc=False),
)
def _gat_edge(hlo, hhi, asrc, adst, src2d, dst2d, out,
              sb2, rows, semr, ss, zb1, den_sh, acc_sh):
    c = lax.axis_index("c")
    s = lax.axis_index("s")
    r0 = s * RPT

    @plsc.parallel_loop(0, RPT, 16)
    def _(i):
        zb1[pl.ds(i, 16)] = jnp.zeros((16,), jnp.float32)

    @plsc.parallel_loop(0, WB, 1)
    def _(k):
        for f in range(DH // 16):
            rows[0][k, pl.ds(f * 16, 16)] = jnp.zeros((16,), jnp.float32)

    pltpu.sync_copy(zb1, den_sh.at[pl.ds(r0, RPT)])
    plsc.subcore_barrier()

    def load_idx(t, b):
        row0 = s * (NSB * NR) + t * NR
        pltpu.async_copy(src2d.at[pl.ds(row0, NR)], b["sidx"], b["si"])
        pltpu.async_copy(dst2d.at[pl.ds(row0, NR)], b["didx"], b["si"])
        pltpu.make_async_copy(src2d.at[pl.ds(row0, NR)], b["sidx"],
                              b["si"]).wait()
        pltpu.make_async_copy(dst2d.at[pl.ds(row0, NR)], b["didx"],
                              b["si"]).wait()

    def p1_start(t, b):
        @pl.when(t >= 2)
        def _():
            for j in range(NR):
                pltpu.make_async_copy(b["al"].at[j],
                                      den_sh.at[b["didx"].at[j]],
                                      b["sp"]).wait()

        load_idx(t, b)
        for j in range(NR):
            pltpu.async_copy(asrc.at[b["sidx"].at[j]], b["va"].at[j], b["sa"])
            pltpu.async_copy(adst.at[b["didx"].at[j]], b["vb"].at[j], b["sb"])

    def p1_finish(b):
        for j in range(NR):
            pltpu.make_async_copy(asrc.at[b["sidx"].at[j]], b["va"].at[j],
                                  b["sa"]).wait()
            pltpu.make_async_copy(adst.at[b["didx"].at[j]], b["vb"].at[j],
                                  b["sb"]).wait()

        for j in range(NR):
            @plsc.parallel_loop(0, KR, 16)
            def _(i):
                e = b["va"][j, pl.ds(i, 16)] + b["vb"][j, pl.ds(i, 16)]
                e = jnp.where(e >= 0, e, 0.2 * e)
                b["al"][j, pl.ds(i, 16)] = jnp.exp(e)

        for j in range(NR):
            pltpu.async_copy(b["al"].at[j], den_sh.at[b["didx"].at[j]],
                             b["sp"], add=True)

    p1_start(0, sb2[0])
    for j in range(RPT // WB):
        pltpu.sync_copy(rows[0].at[pl.ds(0, WB)],
                        acc_sh.at[pl.ds(r0 + j * WB, WB)])

    def p1_pair(p, carry):
        t0 = 2 * p
        p1_start(t0 + 1, sb2[1])
        p1_finish(sb2[0])

        @pl.when(t0 + 2 < NSB)
        def _():
            p1_start(t0 + 2, sb2[0])

        p1_finish(sb2[1])
        return carry

    lax.fori_loop(0, NSB // 2, p1_pair, None)
    for b in sb2:
        for j in range(NR):
            pltpu.make_async_copy(b["al"].at[j], den_sh.at[b["didx"].at[j]],
                                  b["sp"]).wait()
    plsc.subcore_barrier()

    def rowgather(b, j, r):
        @pl.when(c == 0)
        def _():
            pltpu.async_copy(hlo.at[b["sidx"].at[j]], rows[r], semr[r])

        @pl.when(c == 1)
        def _():
            pltpu.async_copy(hhi.at[b["sidx"].at[j]], rows[r], semr[r])

    def p2_start(t, b):
        @pl.when(t >= 2)
        def _():
            for r in range(2):
                pltpu.make_async_copy(rows[r], acc_sh.at[b["didx"].at[0]],
                                      ss[r]).wait()

        load_idx(t, b)
        for j in range(NR):
            pltpu.async_copy(asrc.at[b["sidx"].at[j]], b["va"].at[j], b["sa"])
            pltpu.async_copy(adst.at[b["didx"].at[j]], b["vb"].at[j], b["sb"])
            pltpu.async_copy(den_sh.at[b["didx"].at[j]], b["vd"].at[j],
                             b["sd"])

    def p2_finish(t, b):
        @pl.when(t == NSB - 1)
        def _():
            for r in range(2):
                pltpu.make_async_copy(rows[r], acc_sh.at[b["didx"].at[0]],
                                      ss[r]).wait()

        rowgather(b, 0, 0)
        rowgather(b, 1, 1)

        for j in range(NR):
            pltpu.make_async_copy(asrc.at[b["sidx"].at[j]], b["va"].at[j],
                                  b["sa"]).wait()
            pltpu.make_async_copy(adst.at[b["didx"].at[j]], b["vb"].at[j],
                                  b["sb"]).wait()
            pltpu.make_async_copy(den_sh.at[b["didx"].at[j]], b["vd"].at[j],
                                  b["sd"]).wait()

        for j in range(NR):
            @plsc.parallel_loop(0, KR, 16)
            def _(i):
                e = b["va"][j, pl.ds(i, 16)] + b["vb"][j, pl.ds(i, 16)]
                e = jnp.where(e >= 0, e, 0.2 * e)
                b["al"][j, pl.ds(i, 16)] = (jnp.exp(e) /
                                            (b["vd"][j, pl.ds(i, 16)] + 1e-16))

        for j in range(NR):
            r = j % 2
            pltpu.make_async_copy(hlo.at[b["sidx"].at[j]], rows[r],
                                  semr[r]).wait()
            rbuf, al = rows[r], b["al"]

            @plsc.parallel_loop(0, KR // 16, 1)
            def _(g):
                a16 = al[j, pl.ds(g * 16, 16)]
                for jj in range(16):
                    sp = _splat(a16, jj)
                    k = g * 16 + jj
                    for f in range(DH // 16):
                        rbuf[k, pl.ds(f * 16, 16)] = (
                            rbuf[k, pl.ds(f * 16, 16)] * sp)

            pltpu.async_copy(rbuf, acc_sh.at[b["didx"].at[j]], ss[r],
                             add=True)
            if j + 2 < NR:
                pltpu.make_async_copy(rows[r], acc_sh.at[b["didx"].at[j]],
                                      ss[r]).wait()
                rowgather(b, j + 2, r)

    p2_start(0, sb2[0])

    def p2_pair(p, carry):
        t0 = 2 * p
        p2_start(t0 + 1, sb2[1])
        p2_finish(t0, sb2[0])

        @pl.when(t0 + 2 < NSB)
        def _():
            p2_start(t0 + 2, sb2[0])

        p2_finish(t0 + 1, sb2[1])
        return carry

    lax.fori_loop(0, NSB // 2, p2_pair, None)
    for r in range(2):
        pltpu.make_async_copy(rows[r], acc_sh.at[sb2[1]["didx"].at[0]],
                              ss[r]).wait()
    plsc.subcore_barrier()

    for j in range(RPT // WB):
        pltpu.sync_copy(acc_sh.at[pl.ds(r0 + j * WB, WB)],
                        rows[0].at[pl.ds(0, WB)])
        pltpu.sync_copy(rows[0].at[pl.ds(0, WB)],
                        out.at[c, pl.ds(r0 + j * WB, WB)])


def _elu(v):
    return jnp.where(v > 0, v, jnp.exp(v) - 1.0)


def _dense_first(x_ref, w_ref, ap_ref, hlo_ref, hhi_ref, av_ref):
    h = jnp.dot(x_ref[...], w_ref[...], preferred_element_type=jnp.float32)
    hlo_ref[...] = h[:, :DH]
    hhi_ref[...] = h[:, DH:]
    av_ref[...] = jnp.dot(h, ap_ref[...], preferred_element_type=jnp.float32)


def _dense_mid(p_ref, b_ref, w_ref, ap_ref, hlo_ref, hhi_ref, av_ref):
    v = jnp.concatenate([p_ref[0], p_ref[1]], axis=1)
    v = _elu(v + b_ref[...])
    h = jnp.dot(v, w_ref[...], preferred_element_type=jnp.float32)
    hlo_ref[...] = h[:, :DH]
    hhi_ref[...] = h[:, DH:]
    av_ref[...] = jnp.dot(h, ap_ref[...], preferred_element_type=jnp.float32)


def _dense_last(p_ref, b_ref, o_ref):
    v = jnp.concatenate([p_ref[0], p_ref[1]], axis=1)
    o_ref[...] = _elu(v + b_ref[...])


_BLK = 1000
_G = N // _BLK


def _first(x, W, Apad):
    return pl.pallas_call(
        _dense_first,
        grid=(_G,),
        in_specs=[
            pl.BlockSpec((_BLK, D), lambda i: (i, 0)),
            pl.BlockSpec((D, D), lambda i: (0, 0)),
            pl.BlockSpec((D, D), lambda i: (0, 0)),
        ],
        out_specs=[
            pl.BlockSpec((_BLK, DH), lambda i: (i, 0)),
            pl.BlockSpec((_BLK, DH), lambda i: (i, 0)),
            pl.BlockSpec((_BLK, D), lambda i: (i, 0)),
        ],
        out_shape=[
            jax.ShapeDtypeStruct((N, DH), jnp.float32),
            jax.ShapeDtypeStruct((N, DH), jnp.float32),
            jax.ShapeDtypeStruct((N, D), jnp.float32),
        ],
    )(x, W, Apad)


def _mid(parts, b, W, Apad):
    return pl.pallas_call(
        _dense_mid,
        grid=(_G,),
        in_specs=[
            pl.BlockSpec((NC, _BLK, DH), lambda i: (0, i, 0)),
            pl.BlockSpec((1, D), lambda i: (0, 0)),
            pl.BlockSpec((D, D), lambda i: (0, 0)),
            pl.BlockSpec((D, D), lambda i: (0, 0)),
        ],
        out_specs=[
            pl.BlockSpec((_BLK, DH), lambda i: (i, 0)),
            pl.BlockSpec((_BLK, DH), lambda i: (i, 0)),
            pl.BlockSpec((_BLK, D), lambda i: (i, 0)),
        ],
        out_shape=[
            jax.ShapeDtypeStruct((N, DH), jnp.float32),
            jax.ShapeDtypeStruct((N, DH), jnp.float32),
            jax.ShapeDtypeStruct((N, D), jnp.float32),
        ],
    )(parts, b, W, Apad)


def _last(parts, b):
    return pl.pallas_call(
        _dense_last,
        grid=(_G,),
        in_specs=[
            pl.BlockSpec((NC, _BLK, DH), lambda i: (0, i, 0)),
            pl.BlockSpec((1, D), lambda i: (0, 0)),
        ],
        out_specs=pl.BlockSpec((_BLK, D), lambda i: (i, 0)),
        out_shape=jax.ShapeDtypeStruct((N, D), jnp.float32),
    )(parts, b)


def kernel(x, edge_index, W1, a1_src, a1_dst, b1, W2, a2_src, a2_dst, b2):
    src2d = edge_index[0].reshape(EROWS, KR)
    dst2d = edge_index[1].reshape(EROWS, KR)
    ap1 = jnp.zeros((D, D), jnp.float32).at[:, 0].set(a1_src).at[:, 1].set(a1_dst)
    ap2 = jnp.zeros((D, D), jnp.float32).at[:, 0].set(a2_src).at[:, 1].set(a2_dst)

    hlo1, hhi1, av1 = _first(x, W1, ap1)
    parts1 = _gat_edge(hlo1, hhi1, av1[:, 0], av1[:, 1], src2d, dst2d)
    hlo2, hhi2, av2 = _mid(parts1, b1.reshape(1, D), W2, ap2)
    parts2 = _gat_edge(hlo2, hhi2, av2[:, 0], av2[:, 1], src2d, dst2d)
    return _last(parts2, b2.reshape(1, D))


# speedup vs baseline: 1.1084x; 1.1084x over previous
import functools

import jax
import jax.numpy as jnp
from jax import lax
from jax.experimental import pallas as pl
from jax.experimental.pallas import tpu as pltpu, tpu_sc as plsc

N = 10000
E = 320000
D = 128
DH = D // 2
NPAD = 10240
NC, NS = 2, 16
KR = 400
NR = 5
KSB = NR * KR
NSB = E // NS // KSB
EROWS = E // KR
RPT = NPAD // NS
WB = 320

_mesh = plsc.VectorSubcoreMesh(core_axis_name="c", subcore_axis_name="s",
                               num_cores=NC, num_subcores=NS)


def _splat(v16, j):
    return lax.gather(
        v16, jnp.full((16, 1), j, jnp.int32),
        dimension_numbers=lax.GatherDimensionNumbers(
            offset_dims=(), collapsed_slice_dims=(0,), start_index_map=(0,)),
        slice_sizes=(1,),
        mode=lax.GatherScatterMode.PROMISE_IN_BOUNDS)


@functools.partial(
    pl.kernel,
    out_type=jax.ShapeDtypeStruct((NC, NPAD, DH), jnp.float32),
    mesh=_mesh,
    scratch_types=dict(
        sb2=[dict(sidx=pltpu.VMEM((NR, KR), jnp.int32),
                  didx=pltpu.VMEM((NR, KR), jnp.int32),
                  va=pltpu.VMEM((NR, KR), jnp.float32),
                  vb=pltpu.VMEM((NR, KR), jnp.float32),
                  si=pltpu.SemaphoreType.DMA,
                  sa=pltpu.SemaphoreType.DMA,
                  sb=pltpu.SemaphoreType.DMA,
                  sp=pltpu.SemaphoreType.DMA) for _ in range(2)],
        rows=[pltpu.VMEM((KR, DH), jnp.float32) for _ in range(2)],
        semr=[pltpu.SemaphoreType.DMA for _ in range(2)],
        ss=[pltpu.SemaphoreType.DMA for _ in range(2)],
        zb1=pltpu.VMEM((RPT,), jnp.float32),
        ex_sh=pltpu.VMEM_SHARED((EROWS, KR), jnp.float32),
        den_sh=pltpu.VMEM_SHARED((NPAD,), jnp.float32),
        acc_sh=pltpu.VMEM_SHARED((NPAD, DH), jnp.float32),
    ),
    compiler_params=pltpu.CompilerParams(use_tc_tiling_on_sc=False),
)
def _gat_edge(hlo, hhi, asrc, adst, src2d, dst2d, out,
              sb2, rows, semr, ss, zb1, ex_sh, den_sh, acc_sh):
    c = lax.axis_index("c")
    s = lax.axis_index("s")
    r0 = s * RPT

    @plsc.parallel_loop(0, RPT, 16)
    def _(i):
        zb1[pl.ds(i, 16)] = jnp.zeros((16,), jnp.float32)

    @plsc.parallel_loop(0, WB, 1)
    def _(k):
        for f in range(DH // 16):
            rows[0][k, pl.ds(f * 16, 16)] = jnp.zeros((16,), jnp.float32)

    pltpu.sync_copy(zb1, den_sh.at[pl.ds(r0, RPT)])
    plsc.subcore_barrier()

    def load_idx(t, b):
        row0 = s * (NSB * NR) + t * NR
        pltpu.async_copy(src2d.at[pl.ds(row0, NR)], b["sidx"], b["si"])
        pltpu.async_copy(dst2d.at[pl.ds(row0, NR)], b["didx"], b["si"])
        pltpu.make_async_copy(src2d.at[pl.ds(row0, NR)], b["sidx"],
                              b["si"]).wait()
        pltpu.make_async_copy(dst2d.at[pl.ds(row0, NR)], b["didx"],
                              b["si"]).wait()

    def p1_start(t, b):
        @pl.when(t >= 2)
        def _():
            for j in range(NR):
                pltpu.make_async_copy(b["va"].at[j],
                                      den_sh.at[b["didx"].at[j]],
                                      b["sp"]).wait()
            pltpu.make_async_copy(b["va"], ex_sh.at[pl.ds(0, NR)],
                                  b["sp"]).wait()

        load_idx(t, b)
        for j in range(NR):
            pltpu.async_copy(asrc.at[b["sidx"].at[j]], b["va"].at[j], b["sa"])
            pltpu.async_copy(adst.at[b["didx"].at[j]], b["vb"].at[j], b["sb"])

    def p1_finish(t, b):
        for j in range(NR):
            pltpu.make_async_copy(asrc.at[b["sidx"].at[j]], b["va"].at[j],
                                  b["sa"]).wait()
            pltpu.make_async_copy(adst.at[b["didx"].at[j]], b["vb"].at[j],
                                  b["sb"]).wait()

        for j in range(NR):
            @plsc.parallel_loop(0, KR, 16)
            def _(i):
                e = b["va"][j, pl.ds(i, 16)] + b["vb"][j, pl.ds(i, 16)]
                e = jnp.where(e >= 0, e, 0.2 * e)
                b["va"][j, pl.ds(i, 16)] = jnp.exp(e)

        row0 = s * (NSB * NR) + t * NR
        for j in range(NR):
            pltpu.async_copy(b["va"].at[j], den_sh.at[b["didx"].at[j]],
                             b["sp"], add=True)
        pltpu.async_copy(b["va"], ex_sh.at[pl.ds(row0, NR)], b["sp"])

    p1_start(0, sb2[0])
    for j in range(RPT // WB):
        pltpu.sync_copy(rows[0].at[pl.ds(0, WB)],
                        acc_sh.at[pl.ds(r0 + j * WB, WB)])

    def p1_pair(p, carry):
        t0 = 2 * p
        p1_start(t0 + 1, sb2[1])
        p1_finish(t0, sb2[0])

        @pl.when(t0 + 2 < NSB)
        def _():
            p1_start(t0 + 2, sb2[0])

        p1_finish(t0 + 1, sb2[1])
        return carry

    lax.fori_loop(0, NSB // 2, p1_pair, None)
    for b in sb2:
        for j in range(NR):
            pltpu.make_async_copy(b["va"].at[j], den_sh.at[b["didx"].at[j]],
                                  b["sp"]).wait()
        pltpu.make_async_copy(b["va"], ex_sh.at[pl.ds(0, NR)], b["sp"]).wait()
    plsc.subcore_barrier()

    def rowgather(b, j, r):
        @pl.when(c == 0)
        def _():
            pltpu.async_copy(hlo.at[b["sidx"].at[j]], rows[r], semr[r])

        @pl.when(c == 1)
        def _():
            pltpu.async_copy(hhi.at[b["sidx"].at[j]], rows[r], semr[r])

    def p2_start(t, b):
        @pl.when(t >= 2)
        def _():
            for r in range(2):
                pltpu.make_async_copy(rows[r], acc_sh.at[b["didx"].at[0]],
                                      ss[r]).wait()

        load_idx(t, b)
        row0 = s * (NSB * NR) + t * NR
        pltpu.async_copy(ex_sh.at[pl.ds(row0, NR)], b["va"], b["sa"])
        for j in range(NR):
            pltpu.async_copy(den_sh.at[b["didx"].at[j]], b["vb"].at[j],
                             b["sb"])

    def p2_finish(t, b):
        @pl.when(t == NSB - 1)
        def _():
            for r in range(2):
                pltpu.make_async_copy(rows[r], acc_sh.at[b["didx"].at[0]],
                                      ss[r]).wait()

        rowgather(b, 0, 0)
        rowgather(b, 1, 1)

        pltpu.make_async_copy(ex_sh.at[pl.ds(0, NR)], b["va"], b["sa"]).wait()
        for j in range(NR):
            pltpu.make_async_copy(den_sh.at[b["didx"].at[j]], b["vb"].at[j],
                                  b["sb"]).wait()

        for j in range(NR):
            @plsc.parallel_loop(0, KR, 16)
            def _(i):
                b["va"][j, pl.ds(i, 16)] = (b["va"][j, pl.ds(i, 16)] /
                                            (b["vb"][j, pl.ds(i, 16)] + 1e-16))

        for j in range(NR):
            r = j % 2
            pltpu.make_async_copy(hlo.at[b["sidx"].at[j]], rows[r],
                                  semr[r]).wait()
            rbuf, al = rows[r], b["va"]

            @plsc.parallel_loop(0, KR // 16, 1)
            def _(g):
                a16 = al[j, pl.ds(g * 16, 16)]
                for jj in range(16):
                    sp = _splat(a16, jj)
                    k = g * 16 + jj
                    for f in range(DH // 16):
                        rbuf[k, pl.ds(f * 16, 16)] = (
                            rbuf[k, pl.ds(f * 16, 16)] * sp)

            pltpu.async_copy(rbuf, acc_sh.at[b["didx"].at[j]], ss[r],
                             add=True)
            if j + 2 < NR:
                pltpu.make_async_copy(rows[r], acc_sh.at[b["didx"].at[j]],
                                      ss[r]).wait()
                rowgather(b, j + 2, r)

    p2_start(0, sb2[0])

    def p2_pair(p, carry):
        t0 = 2 * p
        p2_start(t0 + 1, sb2[1])
        p2_finish(t0, sb2[0])

        @pl.when(t0 + 2 < NSB)
        def _():
            p2_start(t0 + 2, sb2[0])

        p2_finish(t0 + 1, sb2[1])
        return carry

    lax.fori_loop(0, NSB // 2, p2_pair, None)
    for r in range(2):
        pltpu.make_async_copy(rows[r], acc_sh.at[sb2[1]["didx"].at[0]],
                              ss[r]).wait()
    plsc.subcore_barrier()

    for j in range(RPT // WB):
        pltpu.sync_copy(acc_sh.at[pl.ds(r0 + j * WB, WB)],
                        rows[0].at[pl.ds(0, WB)])
        pltpu.sync_copy(rows[0].at[pl.ds(0, WB)],
                        out.at[c, pl.ds(r0 + j * WB, WB)])


def _elu(v):
    return jnp.where(v > 0, v, jnp.exp(v) - 1.0)


def _dense_first(x_ref, w_ref, ap_ref, hlo_ref, hhi_ref, av_ref):
    h = jnp.dot(x_ref[...], w_ref[...], preferred_element_type=jnp.float32)
    hlo_ref[...] = h[:, :DH]
    hhi_ref[...] = h[:, DH:]
    av_ref[...] = jnp.dot(h, ap_ref[...], preferred_element_type=jnp.float32)


def _dense_mid(p_ref, b_ref, w_ref, ap_ref, hlo_ref, hhi_ref, av_ref):
    v = jnp.concatenate([p_ref[0], p_ref[1]], axis=1)
    v = _elu(v + b_ref[...])
    h = jnp.dot(v, w_ref[...], preferred_element_type=jnp.float32)
    hlo_ref[...] = h[:, :DH]
    hhi_ref[...] = h[:, DH:]
    av_ref[...] = jnp.dot(h, ap_ref[...], preferred_element_type=jnp.float32)


def _dense_last(p_ref, b_ref, o_ref):
    v = jnp.concatenate([p_ref[0], p_ref[1]], axis=1)
    o_ref[...] = _elu(v + b_ref[...])


_BLK = 1000
_G = N // _BLK


def _first(x, W, Apad):
    return pl.pallas_call(
        _dense_first,
        grid=(_G,),
        in_specs=[
            pl.BlockSpec((_BLK, D), lambda i: (i, 0)),
            pl.BlockSpec((D, D), lambda i: (0, 0)),
            pl.BlockSpec((D, D), lambda i: (0, 0)),
        ],
        out_specs=[
            pl.BlockSpec((_BLK, DH), lambda i: (i, 0)),
            pl.BlockSpec((_BLK, DH), lambda i: (i, 0)),
            pl.BlockSpec((_BLK, D), lambda i: (i, 0)),
        ],
        out_shape=[
            jax.ShapeDtypeStruct((N, DH), jnp.float32),
            jax.ShapeDtypeStruct((N, DH), jnp.float32),
            jax.ShapeDtypeStruct((N, D), jnp.float32),
        ],
    )(x, W, Apad)


def _mid(parts, b, W, Apad):
    return pl.pallas_call(
        _dense_mid,
        grid=(_G,),
        in_specs=[
            pl.BlockSpec((NC, _BLK, DH), lambda i: (0, i, 0)),
            pl.BlockSpec((1, D), lambda i: (0, 0)),
            pl.BlockSpec((D, D), lambda i: (0, 0)),
            pl.BlockSpec((D, D), lambda i: (0, 0)),
        ],
        out_specs=[
            pl.BlockSpec((_BLK, DH), lambda i: (i, 0)),
            pl.BlockSpec((_BLK, DH), lambda i: (i, 0)),
            pl.BlockSpec((_BLK, D), lambda i: (i, 0)),
        ],
        out_shape=[
            jax.ShapeDtypeStruct((N, DH), jnp.float32),
            jax.ShapeDtypeStruct((N, DH), jnp.float32),
            jax.ShapeDtypeStruct((N, D), jnp.float32),
        ],
    )(parts, b, W, Apad)


def _last(parts, b):
    return pl.pallas_call(
        _dense_last,
        grid=(_G,),
        in_specs=[
            pl.BlockSpec((NC, _BLK, DH), lambda i: (0, i, 0)),
            pl.BlockSpec((1, D), lambda i: (0, 0)),
        ],
        out_specs=pl.BlockSpec((_BLK, D), lambda i: (i, 0)),
        out_shape=jax.ShapeDtypeStruct((N, D), jnp.float32),
    )(parts, b)


def kernel(x, edge_index, W1, a1_src, a1_dst, b1, W2, a2_src, a2_dst, b2):
    src2d = edge_index[0].reshape(EROWS, KR)
    dst2d = edge_index[1].reshape(EROWS, KR)
    ap1 = jnp.zeros((D, D), jnp.float32).at[:, 0].set(a1_src).at[:, 1].set(a1_dst)
    ap2 = jnp.zeros((D, D), jnp.float32).at[:, 0].set(a2_src).at[:, 1].set(a2_dst)

    hlo1, hhi1, av1 = _first(x, W1, ap1)
    parts1 = _gat_edge(hlo1, hhi1, av1[:, 0], av1[:, 1], src2d, dst2d)
    hlo2, hhi2, av2 = _mid(parts1, b1.reshape(1, D), W2, ap2)
    parts2 = _gat_edge(hlo2, hhi2, av2[:, 0], av2[:, 1], src2d, dst2d)
    return _last(parts2, b2.reshape(1, D))


# speedup vs baseline: 1.1283x; 1.0180x over previous
import functools

import jax
import jax.numpy as jnp
from jax import lax
from jax.experimental import pallas as pl
from jax.experimental.pallas import tpu as pltpu, tpu_sc as plsc

N = 10000
E = 320000
D = 128
DH = D // 2
NPAD = 10240
NC, NS = 2, 16
KR = 400
NR = 5
KSB = NR * KR
NSB = E // NS // KSB
EROWS = E // KR
RPT = NPAD // NS
WB = 320

_mesh = plsc.VectorSubcoreMesh(core_axis_name="c", subcore_axis_name="s",
                               num_cores=NC, num_subcores=NS)


def _splat(v16, j):
    return lax.gather(
        v16, jnp.full((16, 1), j, jnp.int32),
        dimension_numbers=lax.GatherDimensionNumbers(
            offset_dims=(), collapsed_slice_dims=(0,), start_index_map=(0,)),
        slice_sizes=(1,),
        mode=lax.GatherScatterMode.PROMISE_IN_BOUNDS)


@functools.partial(
    pl.kernel,
    out_type=jax.ShapeDtypeStruct((NC, NPAD, DH), jnp.float32),
    mesh=_mesh,
    scratch_types=dict(
        sb2=[dict(sidx=pltpu.VMEM((NR, KR), jnp.int32),
                  didx=pltpu.VMEM((NR, KR), jnp.int32),
                  va=pltpu.VMEM((NR, KR), jnp.float32),
                  vb=pltpu.VMEM((NR, KR), jnp.float32),
                  si=pltpu.SemaphoreType.DMA,
                  sa=pltpu.SemaphoreType.DMA,
                  sb=pltpu.SemaphoreType.DMA,
                  sp=pltpu.SemaphoreType.DMA) for _ in range(2)],
        rows=[pltpu.VMEM((KR, DH), jnp.float32) for _ in range(2)],
        semr=[pltpu.SemaphoreType.DMA for _ in range(2)],
        ss=[pltpu.SemaphoreType.DMA for _ in range(2)],
        zb1=pltpu.VMEM((RPT,), jnp.float32),
        ex_sh=pltpu.VMEM_SHARED((EROWS, KR), jnp.float32),
        den_sh=pltpu.VMEM_SHARED((NPAD,), jnp.float32),
        acc_sh=pltpu.VMEM_SHARED((NPAD, DH), jnp.float32),
    ),
    compiler_params=pltpu.CompilerParams(use_tc_tiling_on_sc=False),
)
def _gat_edge(hlo, hhi, asrc, adst, src2d, dst2d, out,
              sb2, rows, semr, ss, zb1, ex_sh, den_sh, acc_sh):
    c = lax.axis_index("c")
    s = lax.axis_index("s")
    r0 = s * RPT

    @plsc.parallel_loop(0, RPT, 16)
    def _(i):
        zb1[pl.ds(i, 16)] = jnp.zeros((16,), jnp.float32)

    @plsc.parallel_loop(0, WB, 1)
    def _(k):
        for f in range(DH // 16):
            rows[0][k, pl.ds(f * 16, 16)] = jnp.zeros((16,), jnp.float32)

    pltpu.sync_copy(zb1, den_sh.at[pl.ds(r0, RPT)])
    plsc.subcore_barrier()

    def load_idx(t, b):
        row0 = s * (NSB * NR) + t * NR
        pltpu.async_copy(src2d.at[pl.ds(row0, NR)], b["sidx"], b["si"])
        pltpu.async_copy(dst2d.at[pl.ds(row0, NR)], b["didx"], b["si"])
        pltpu.make_async_copy(src2d.at[pl.ds(row0, NR)], b["sidx"],
                              b["si"]).wait()
        pltpu.make_async_copy(dst2d.at[pl.ds(row0, NR)], b["didx"],
                              b["si"]).wait()

    def p1_start(t, b):
        @pl.when(t >= 2)
        def _():
            for j in range(NR):
                pltpu.make_async_copy(b["va"].at[j],
                                      den_sh.at[b["didx"].at[j]],
                                      b["sp"]).wait()
            pltpu.make_async_copy(b["va"], ex_sh.at[pl.ds(0, NR)],
                                  b["sp"]).wait()

        load_idx(t, b)
        for j in range(NR):
            pltpu.async_copy(asrc.at[b["sidx"].at[j]], b["va"].at[j], b["sa"])
            pltpu.async_copy(adst.at[b["didx"].at[j]], b["vb"].at[j], b["sb"])

    def p1_finish(t, b):
        for j in range(NR):
            pltpu.make_async_copy(asrc.at[b["sidx"].at[j]], b["va"].at[j],
                                  b["sa"]).wait()
            pltpu.make_async_copy(adst.at[b["didx"].at[j]], b["vb"].at[j],
                                  b["sb"]).wait()

        for j in range(NR):
            @plsc.parallel_loop(0, KR, 16)
            def _(i):
                e = b["va"][j, pl.ds(i, 16)] + b["vb"][j, pl.ds(i, 16)]
                e = jnp.where(e >= 0, e, 0.2 * e)
                b["va"][j, pl.ds(i, 16)] = jnp.exp(e)

        row0 = s * (NSB * NR) + t * NR
        for j in range(NR):
            pltpu.async_copy(b["va"].at[j], den_sh.at[b["didx"].at[j]],
                             b["sp"], add=True)
        pltpu.async_copy(b["va"], ex_sh.at[pl.ds(row0, NR)], b["sp"])

    p1_start(0, sb2[0])
    for j in range(RPT // WB):
        pltpu.sync_copy(rows[0].at[pl.ds(0, WB)],
                        acc_sh.at[pl.ds(r0 + j * WB, WB)])

    def p1_pair(p, carry):
        t0 = 2 * p
        p1_start(t0 + 1, sb2[1])
        p1_finish(t0, sb2[0])

        @pl.when(t0 + 2 < NSB)
        def _():
            p1_start(t0 + 2, sb2[0])

        p1_finish(t0 + 1, sb2[1])
        return carry

    lax.fori_loop(0, NSB // 2, p1_pair, None)
    for b in sb2:
        for j in range(NR):
            pltpu.make_async_copy(b["va"].at[j], den_sh.at[b["didx"].at[j]],
                                  b["sp"]).wait()
        pltpu.make_async_copy(b["va"], ex_sh.at[pl.ds(0, NR)], b["sp"]).wait()
    plsc.subcore_barrier()

    def rowgather(b, j, r):
        @pl.when(c == 0)
        def _():
            pltpu.async_copy(hlo.at[b["sidx"].at[j]], rows[r], semr[r])

        @pl.when(c == 1)
        def _():
            pltpu.async_copy(hhi.at[b["sidx"].at[j]], rows[r], semr[r])

    def p2_start(t, b):
        @pl.when(t >= 2)
        def _():
            for r in range(2):
                pltpu.make_async_copy(rows[r], acc_sh.at[b["didx"].at[0]],
                                      ss[r]).wait()

        load_idx(t, b)
        row0 = s * (NSB * NR) + t * NR
        pltpu.async_copy(ex_sh.at[pl.ds(row0, NR)], b["va"], b["sa"])

    def p2_finish(t, b):
        @pl.when(t == NSB - 1)
        def _():
            for r in range(2):
                pltpu.make_async_copy(rows[r], acc_sh.at[b["didx"].at[0]],
                                      ss[r]).wait()

        rowgather(b, 0, 0)
        rowgather(b, 1, 1)

        pltpu.make_async_copy(ex_sh.at[pl.ds(0, NR)], b["va"], b["sa"]).wait()

        for j in range(NR):
            r = j % 2
            pltpu.make_async_copy(hlo.at[b["sidx"].at[j]], rows[r],
                                  semr[r]).wait()
            rbuf, al = rows[r], b["va"]

            @plsc.parallel_loop(0, KR // 16, 1)
            def _(g):
                a16 = al[j, pl.ds(g * 16, 16)]
                for jj in range(16):
                    sp = _splat(a16, jj)
                    k = g * 16 + jj
                    for f in range(DH // 16):
                        rbuf[k, pl.ds(f * 16, 16)] = (
                            rbuf[k, pl.ds(f * 16, 16)] * sp)

            pltpu.async_copy(rbuf, acc_sh.at[b["didx"].at[j]], ss[r],
                             add=True)
            if j + 2 < NR:
                pltpu.make_async_copy(rows[r], acc_sh.at[b["didx"].at[j]],
                                      ss[r]).wait()
                rowgather(b, j + 2, r)

    p2_start(0, sb2[0])

    def p2_pair(p, carry):
        t0 = 2 * p
        p2_start(t0 + 1, sb2[1])
        p2_finish(t0, sb2[0])

        @pl.when(t0 + 2 < NSB)
        def _():
            p2_start(t0 + 2, sb2[0])

        p2_finish(t0 + 1, sb2[1])
        return carry

    lax.fori_loop(0, NSB // 2, p2_pair, None)
    for r in range(2):
        pltpu.make_async_copy(rows[r], acc_sh.at[sb2[1]["didx"].at[0]],
                              ss[r]).wait()
    plsc.subcore_barrier()

    for j in range(RPT // WB):
        pltpu.sync_copy(acc_sh.at[pl.ds(r0 + j * WB, WB)],
                        rows[0].at[pl.ds(0, WB)])
        pltpu.sync_copy(den_sh.at[pl.ds(r0 + j * WB, WB)],
                        zb1.at[pl.ds(0, WB)])
        rbuf = rows[0]

        @plsc.parallel_loop(0, WB // 16, 1)
        def _(g):
            d16 = 1.0 / (zb1[pl.ds(g * 16, 16)] + 1e-16)
            for jj in range(16):
                sp = _splat(d16, jj)
                k = g * 16 + jj
                for f in range(DH // 16):
                    rbuf[k, pl.ds(f * 16, 16)] = rbuf[k, pl.ds(f * 16, 16)] * sp

        pltpu.sync_copy(rows[0].at[pl.ds(0, WB)],
                        out.at[c, pl.ds(r0 + j * WB, WB)])


def _elu(v):
    return jnp.where(v > 0, v, jnp.exp(v) - 1.0)


def _dense_first(x_ref, w_ref, ap_ref, hlo_ref, hhi_ref, av_ref):
    h = jnp.dot(x_ref[...], w_ref[...], preferred_element_type=jnp.float32)
    hlo_ref[...] = h[:, :DH]
    hhi_ref[...] = h[:, DH:]
    av_ref[...] = jnp.dot(h, ap_ref[...], preferred_element_type=jnp.float32)


def _dense_mid(p_ref, b_ref, w_ref, ap_ref, hlo_ref, hhi_ref, av_ref):
    v = jnp.concatenate([p_ref[0], p_ref[1]], axis=1)
    v = _elu(v + b_ref[...])
    h = jnp.dot(v, w_ref[...], preferred_element_type=jnp.float32)
    hlo_ref[...] = h[:, :DH]
    hhi_ref[...] = h[:, DH:]
    av_ref[...] = jnp.dot(h, ap_ref[...], preferred_element_type=jnp.float32)


def _dense_last(p_ref, b_ref, o_ref):
    v = jnp.concatenate([p_ref[0], p_ref[1]], axis=1)
    o_ref[...] = _elu(v + b_ref[...])


_BLK = 1000
_G = N // _BLK


def _first(x, W, Apad):
    return pl.pallas_call(
        _dense_first,
        grid=(_G,),
        in_specs=[
            pl.BlockSpec((_BLK, D), lambda i: (i, 0)),
            pl.BlockSpec((D, D), lambda i: (0, 0)),
            pl.BlockSpec((D, D), lambda i: (0, 0)),
        ],
        out_specs=[
            pl.BlockSpec((_BLK, DH), lambda i: (i, 0)),
            pl.BlockSpec((_BLK, DH), lambda i: (i, 0)),
            pl.BlockSpec((_BLK, D), lambda i: (i, 0)),
        ],
        out_shape=[
            jax.ShapeDtypeStruct((N, DH), jnp.float32),
            jax.ShapeDtypeStruct((N, DH), jnp.float32),
            jax.ShapeDtypeStruct((N, D), jnp.float32),
        ],
    )(x, W, Apad)


def _mid(parts, b, W, Apad):
    return pl.pallas_call(
        _dense_mid,
        grid=(_G,),
        in_specs=[
            pl.BlockSpec((NC, _BLK, DH), lambda i: (0, i, 0)),
            pl.BlockSpec((1, D), lambda i: (0, 0)),
            pl.BlockSpec((D, D), lambda i: (0, 0)),
            pl.BlockSpec((D, D), lambda i: (0, 0)),
        ],
        out_specs=[
            pl.BlockSpec((_BLK, DH), lambda i: (i, 0)),
            pl.BlockSpec((_BLK, DH), lambda i: (i, 0)),
            pl.BlockSpec((_BLK, D), lambda i: (i, 0)),
        ],
        out_shape=[
            jax.ShapeDtypeStruct((N, DH), jnp.float32),
            jax.ShapeDtypeStruct((N, DH), jnp.float32),
            jax.ShapeDtypeStruct((N, D), jnp.float32),
        ],
    )(parts, b, W, Apad)


def _last(parts, b):
    return pl.pallas_call(
        _dense_last,
        grid=(_G,),
        in_specs=[
            pl.BlockSpec((NC, _BLK, DH), lambda i: (0, i, 0)),
            pl.BlockSpec((1, D), lambda i: (0, 0)),
        ],
        out_specs=pl.BlockSpec((_BLK, D), lambda i: (i, 0)),
        out_shape=jax.ShapeDtypeStruct((N, D), jnp.float32),
    )(parts, b)


def kernel(x, edge_index, W1, a1_src, a1_dst, b1, W2, a2_src, a2_dst, b2):
    src2d = edge_index[0].reshape(EROWS, KR)
    dst2d = edge_index[1].reshape(EROWS, KR)
    ap1 = jnp.zeros((D, D), jnp.float32).at[:, 0].set(a1_src).at[:, 1].set(a1_dst)
    ap2 = jnp.zeros((D, D), jnp.float32).at[:, 0].set(a2_src).at[:, 1].set(a2_dst)

    hlo1, hhi1, av1 = _first(x, W1, ap1)
    parts1 = _gat_edge(hlo1, hhi1, av1[:, 0], av1[:, 1], src2d, dst2d)
    hlo2, hhi2, av2 = _mid(parts1, b1.reshape(1, D), W2, ap2)
    parts2 = _gat_edge(hlo2, hhi2, av2[:, 0], av2[:, 1], src2d, dst2d)
    return _last(parts2, b2.reshape(1, D))


# speedup vs baseline: 1.1289x; 1.0006x over previous
import functools

import jax
import jax.numpy as jnp
from jax import lax
from jax.experimental import pallas as pl
from jax.experimental.pallas import tpu as pltpu, tpu_sc as plsc

N = 10000
E = 320000
D = 128
DH = D // 2
NPAD = 10240
NC, NS = 2, 16
KR = 400
NR = 5
KSB = NR * KR
NSB = E // NS // KSB
EROWS = E // KR
RPT = NPAD // NS
WB = 320

_mesh = plsc.VectorSubcoreMesh(core_axis_name="c", subcore_axis_name="s",
                               num_cores=NC, num_subcores=NS)


def _splat(v16, j):
    return lax.gather(
        v16, jnp.full((16, 1), j, jnp.int32),
        dimension_numbers=lax.GatherDimensionNumbers(
            offset_dims=(), collapsed_slice_dims=(0,), start_index_map=(0,)),
        slice_sizes=(1,),
        mode=lax.GatherScatterMode.PROMISE_IN_BOUNDS)


@functools.partial(
    pl.kernel,
    out_type=jax.ShapeDtypeStruct((NC, NPAD, DH), jnp.float32),
    mesh=_mesh,
    scratch_types=dict(
        sb2=[dict(sidx=pltpu.VMEM((NR, KR), jnp.int32),
                  didx=pltpu.VMEM((NR, KR), jnp.int32),
                  va=pltpu.VMEM((NR, KR), jnp.float32),
                  vb=pltpu.VMEM((NR, KR), jnp.float32),
                  si=pltpu.SemaphoreType.DMA,
                  sa=pltpu.SemaphoreType.DMA,
                  sb=pltpu.SemaphoreType.DMA,
                  sp=pltpu.SemaphoreType.DMA) for _ in range(2)],
        rows=[pltpu.VMEM((KR, DH), jnp.float32) for _ in range(2)],
        semr=[pltpu.SemaphoreType.DMA for _ in range(2)],
        ss=[pltpu.SemaphoreType.DMA for _ in range(2)],
        zb1=pltpu.VMEM((RPT,), jnp.float32),
        ex_sh=pltpu.VMEM_SHARED((EROWS, KR), jnp.float32),
        den_sh=pltpu.VMEM_SHARED((NPAD,), jnp.float32),
        acc_sh=pltpu.VMEM_SHARED((NPAD, DH), jnp.float32),
    ),
    compiler_params=pltpu.CompilerParams(use_tc_tiling_on_sc=False),
)
def _gat_edge(hlo, hhi, asrc, adst, src2d, dst2d, out,
              sb2, rows, semr, ss, zb1, ex_sh, den_sh, acc_sh):
    c = lax.axis_index("c")
    s = lax.axis_index("s")
    r0 = s * RPT

    @plsc.parallel_loop(0, RPT, 16)
    def _(i):
        zb1[pl.ds(i, 16)] = jnp.zeros((16,), jnp.float32)

    @plsc.parallel_loop(0, WB, 1)
    def _(k):
        for f in range(DH // 16):
            rows[0][k, pl.ds(f * 16, 16)] = jnp.zeros((16,), jnp.float32)

    pltpu.sync_copy(zb1, den_sh.at[pl.ds(r0, RPT)])
    plsc.subcore_barrier()

    def load_idx(t, b):
        row0 = s * (NSB * NR) + t * NR
        pltpu.async_copy(src2d.at[pl.ds(row0, NR)], b["sidx"], b["si"])
        pltpu.async_copy(dst2d.at[pl.ds(row0, NR)], b["didx"], b["si"])
        pltpu.make_async_copy(src2d.at[pl.ds(row0, NR)], b["sidx"],
                              b["si"]).wait()
        pltpu.make_async_copy(dst2d.at[pl.ds(row0, NR)], b["didx"],
                              b["si"]).wait()

    def p1_start(t, b):
        @pl.when(t >= 2)
        def _():
            for j in range(NR):
                pltpu.make_async_copy(b["va"].at[j],
                                      den_sh.at[b["didx"].at[j]],
                                      b["sp"]).wait()
            pltpu.make_async_copy(b["va"], ex_sh.at[pl.ds(0, NR)],
                                  b["sp"]).wait()

        load_idx(t, b)
        for j in range(NR):
            pltpu.async_copy(asrc.at[b["sidx"].at[j]], b["va"].at[j], b["sa"])
            pltpu.async_copy(adst.at[b["didx"].at[j]], b["vb"].at[j], b["sb"])

    def p1_finish(t, b):
        for j in range(NR):
            pltpu.make_async_copy(asrc.at[b["sidx"].at[j]], b["va"].at[j],
                                  b["sa"]).wait()
            pltpu.make_async_copy(adst.at[b["didx"].at[j]], b["vb"].at[j],
                                  b["sb"]).wait()

        for j in range(NR):
            @plsc.parallel_loop(0, KR, 16)
            def _(i):
                e = b["va"][j, pl.ds(i, 16)] + b["vb"][j, pl.ds(i, 16)]
                e = jnp.where(e >= 0, e, 0.2 * e)
                b["va"][j, pl.ds(i, 16)] = jnp.exp(e)

        row0 = s * (NSB * NR) + t * NR
        for j in range(NR):
            pltpu.async_copy(b["va"].at[j], den_sh.at[b["didx"].at[j]],
                             b["sp"], add=True)
        pltpu.async_copy(b["va"], ex_sh.at[pl.ds(row0, NR)], b["sp"])

    p1_start(0, sb2[0])
    for j in range(RPT // WB):
        pltpu.sync_copy(rows[0].at[pl.ds(0, WB)],
                        acc_sh.at[pl.ds(r0 + j * WB, WB)])

    def p1_pair(p, carry):
        t0 = 2 * p
        p1_start(t0 + 1, sb2[1])
        p1_finish(t0, sb2[0])

        @pl.when(t0 + 2 < NSB)
        def _():
            p1_start(t0 + 2, sb2[0])

        p1_finish(t0 + 1, sb2[1])
        return carry

    lax.fori_loop(0, NSB // 2, p1_pair, None)
    for b in sb2:
        for j in range(NR):
            pltpu.make_async_copy(b["va"].at[j], den_sh.at[b["didx"].at[j]],
                                  b["sp"]).wait()
        pltpu.make_async_copy(b["va"], ex_sh.at[pl.ds(0, NR)], b["sp"]).wait()
    plsc.subcore_barrier()

    def rowgather(b, j, r):
        @pl.when(c == 0)
        def _():
            pltpu.async_copy(hlo.at[b["sidx"].at[j]], rows[r], semr[r])

        @pl.when(c == 1)
        def _():
            pltpu.async_copy(hhi.at[b["sidx"].at[j]], rows[r], semr[r])

    def p2_start(t, b):
        @pl.when(t >= 2)
        def _():
            for r in range(2):
                pltpu.make_async_copy(rows[r], acc_sh.at[b["didx"].at[0]],
                                      ss[r]).wait()

        load_idx(t, b)
        row0 = s * (NSB * NR) + t * NR
        pltpu.async_copy(ex_sh.at[pl.ds(row0, NR)], b["va"], b["sa"])

    def p2_finish(t, b):
        @pl.when(t == NSB - 1)
        def _():
            for r in range(2):
                pltpu.make_async_copy(rows[r], acc_sh.at[b["didx"].at[0]],
                                      ss[r]).wait()

        rowgather(b, 0, 0)
        rowgather(b, 1, 1)

        pltpu.make_async_copy(ex_sh.at[pl.ds(0, NR)], b["va"], b["sa"]).wait()

        for j in range(NR):
            r = j % 2
            pltpu.make_async_copy(hlo.at[b["sidx"].at[j]], rows[r],
                                  semr[r]).wait()
            rbuf, al = rows[r], b["va"]

            @plsc.parallel_loop(0, KR // 16, 1)
            def _(g):
                a16 = al[j, pl.ds(g * 16, 16)]
                for jj in range(16):
                    sp = _splat(a16, jj)
                    k = g * 16 + jj
                    for f in range(DH // 16):
                        rbuf[k, pl.ds(f * 16, 16)] = (
                            rbuf[k, pl.ds(f * 16, 16)] * sp)

            pltpu.async_copy(rbuf, acc_sh.at[b["didx"].at[j]], ss[r],
                             add=True)
            if j + 2 < NR:
                pltpu.make_async_copy(rows[r], acc_sh.at[b["didx"].at[j]],
                                      ss[r]).wait()
                rowgather(b, j + 2, r)

    p2_start(0, sb2[0])

    def p2_pair(p, carry):
        t0 = 2 * p
        p2_start(t0 + 1, sb2[1])
        p2_finish(t0, sb2[0])

        @pl.when(t0 + 2 < NSB)
        def _():
            p2_start(t0 + 2, sb2[0])

        p2_finish(t0 + 1, sb2[1])
        return carry

    lax.fori_loop(0, NSB // 2, p2_pair, None)
    for r in range(2):
        pltpu.make_async_copy(rows[r], acc_sh.at[sb2[1]["didx"].at[0]],
                              ss[r]).wait()
    plsc.subcore_barrier()

    for j in range(RPT // WB):
        pltpu.sync_copy(acc_sh.at[pl.ds(r0 + j * WB, WB)],
                        rows[0].at[pl.ds(0, WB)])
        pltpu.sync_copy(den_sh.at[pl.ds(r0 + j * WB, WB)],
                        zb1.at[pl.ds(0, WB)])
        rbuf = rows[0]

        @plsc.parallel_loop(0, WB // 16, 1)
        def _(g):
            d16 = 1.0 / (zb1[pl.ds(g * 16, 16)] + 1e-16)
            for jj in range(16):
                sp = _splat(d16, jj)
                k = g * 16 + jj
                for f in range(DH // 16):
                    rbuf[k, pl.ds(f * 16, 16)] = rbuf[k, pl.ds(f * 16, 16)] * sp

        pltpu.sync_copy(rows[0].at[pl.ds(0, WB)],
                        out.at[c, pl.ds(r0 + j * WB, WB)])


def _elu(v):
    return jnp.where(v > 0, v, jnp.exp(v) - 1.0)


def _dense_first(x_ref, w_ref, ap_ref, hlo_ref, hhi_ref, av_ref):
    h = jnp.dot(x_ref[...], w_ref[...], preferred_element_type=jnp.float32)
    hlo_ref[...] = h[:, :DH]
    hhi_ref[...] = h[:, DH:]
    av_ref[...] = jnp.dot(h, ap_ref[...], preferred_element_type=jnp.float32)


def _dense_mid(p_ref, b_ref, w_ref, ap_ref, hlo_ref, hhi_ref, av_ref):
    v = jnp.concatenate([p_ref[0], p_ref[1]], axis=1)
    v = _elu(v + b_ref[...])
    h = jnp.dot(v, w_ref[...], preferred_element_type=jnp.float32)
    hlo_ref[...] = h[:, :DH]
    hhi_ref[...] = h[:, DH:]
    av_ref[...] = jnp.dot(h, ap_ref[...], preferred_element_type=jnp.float32)


def _dense_last(p_ref, b_ref, o_ref):
    v = jnp.concatenate([p_ref[0], p_ref[1]], axis=1)
    o_ref[...] = _elu(v + b_ref[...])


_BLK = 1000
_G = N // _BLK


_DENSE_OUT = [
    jax.ShapeDtypeStruct((N, DH), jnp.float32),
    jax.ShapeDtypeStruct((N, DH), jnp.float32),
    jax.ShapeDtypeStruct((N, D), jnp.float32),
]
_DENSE_OUT_SPECS = [
    pl.BlockSpec((_BLK, DH), lambda i: (i, 0)),
    pl.BlockSpec((_BLK, DH), lambda i: (i, 0)),
    pl.BlockSpec((_BLK, D), lambda i: (i, 0)),
]


def _first(x, W, Apad):
    return pl.pallas_call(
        _dense_first,
        grid=(_G,),
        in_specs=[
            pl.BlockSpec((_BLK, D), lambda i: (i, 0)),
            pl.BlockSpec((D, D), lambda i: (0, 0)),
            pl.BlockSpec((D, D), lambda i: (0, 0)),
        ],
        out_specs=_DENSE_OUT_SPECS,
        out_shape=_DENSE_OUT,
    )(x, W, Apad)


def _mid(parts, b, W, Apad):
    return pl.pallas_call(
        _dense_mid,
        grid=(_G,),
        in_specs=[
            pl.BlockSpec((NC, _BLK, DH), lambda i: (0, i, 0)),
            pl.BlockSpec((1, D), lambda i: (0, 0)),
            pl.BlockSpec((D, D), lambda i: (0, 0)),
            pl.BlockSpec((D, D), lambda i: (0, 0)),
        ],
        out_specs=_DENSE_OUT_SPECS,
        out_shape=_DENSE_OUT,
    )(parts, b, W, Apad)


def _last(parts, b):
    return pl.pallas_call(
        _dense_last,
        grid=(_G,),
        in_specs=[
            pl.BlockSpec((NC, _BLK, DH), lambda i: (0, i, 0)),
            pl.BlockSpec((1, D), lambda i: (0, 0)),
        ],
        out_specs=pl.BlockSpec((_BLK, D), lambda i: (i, 0)),
        out_shape=jax.ShapeDtypeStruct((N, D), jnp.float32),
    )(parts, b)


def kernel(x, edge_index, W1, a1_src, a1_dst, b1, W2, a2_src, a2_dst, b2):
    src2d = edge_index[0].reshape(EROWS, KR)
    dst2d = edge_index[1].reshape(EROWS, KR)
    ap1 = jnp.zeros((D, D), jnp.float32).at[:, 0].set(a1_src).at[:, 1].set(a1_dst)
    ap2 = jnp.zeros((D, D), jnp.float32).at[:, 0].set(a2_src).at[:, 1].set(a2_dst)

    hlo1, hhi1, av1 = _first(x, W1, ap1)
    parts1 = _gat_edge(hlo1, hhi1, av1[:, 0], av1[:, 1], src2d, dst2d)
    hlo2, hhi2, av2 = _mid(parts1, b1.reshape(1, D), W2, ap2)
    parts2 = _gat_edge(hlo2, hhi2, av2[:, 0], av2[:, 1], src2d, dst2d)
    return _last(parts2, b2.reshape(1, D))
